# baseline (speedup 1.0000x reference)
import functools

import jax
import jax.numpy as jnp
from jax import lax
from jax.experimental import pallas as pl
from jax.experimental.pallas import tpu as pltpu
from jax.experimental.pallas import tpu_sc as plsc

B = 16384
D = 32
NC = 2
NS = 16
NW = NC * NS
BPW = B // NW
CH = 128
NCHUNK = BPW // CH


def _sc_gather(user_idx, item_idx, embed_user_w, embed_item_w):
    mesh = plsc.VectorSubcoreMesh(core_axis_name="c", subcore_axis_name="s")

    @functools.partial(
        pl.kernel,
        mesh=mesh,
        compiler_params=pltpu.CompilerParams(use_tc_tiling_on_sc=False),
        out_type=(
            jax.ShapeDtypeStruct((B, D), jnp.bfloat16),
            jax.ShapeDtypeStruct((B, D), jnp.bfloat16),
        ),
        scratch_types=[
            pltpu.VMEM((NCHUNK, CH), jnp.int32),
            pltpu.VMEM((NCHUNK, CH), jnp.int32),
            pltpu.VMEM((BPW, D), jnp.bfloat16),
            pltpu.VMEM((BPW, D), jnp.bfloat16),
            pltpu.SemaphoreType.DMA,
            pltpu.SemaphoreType.DMA,
        ],
    )
    def k(uw_hbm, iw_hbm, uidx_hbm, iidx_hbm, eu_hbm, ei_hbm,
          uidx_v, iidx_v, urows_v, irows_v, sem_u, sem_i):
        wid = lax.axis_index("s") * NC + lax.axis_index("c")
        base = wid * BPW
        pltpu.sync_copy(uidx_hbm.at[wid], uidx_v)
        pltpu.sync_copy(iidx_hbm.at[wid], iidx_v)
        copies = []
        for j in range(NCHUNK):
            dst = pl.ds(j * CH, CH)
            copies.append(
                pltpu.async_copy(uw_hbm.at[uidx_v.at[j]], urows_v.at[dst], sem_u))
            copies.append(
                pltpu.async_copy(iw_hbm.at[iidx_v.at[j]], irows_v.at[dst], sem_i))
        for c in copies:
            c.wait()
        pltpu.sync_copy(urows_v, eu_hbm.at[pl.ds(base, BPW)])
        pltpu.sync_copy(irows_v, ei_hbm.at[pl.ds(base, BPW)])

    return k(embed_user_w, embed_item_w,
             user_idx.reshape(NW, NCHUNK, CH),
             item_idx.reshape(NW, NCHUNK, CH))


def _tc_loss_body(eu_ref, ei_ref, lab_ref, w_ref, b_ref, out_ref):
    eu = eu_ref[...].astype(jnp.float32)
    ei = ei_ref[...].astype(jnp.float32)
    t = eu * ei * w_ref[...]
    x = jnp.sum(t, axis=1) + b_ref[0]
    y = lab_ref[...]
    terms = jnp.maximum(x, 0.0) - x * y + jnp.log1p(jnp.exp(-jnp.abs(x)))
    out_ref[...] = (jnp.sum(terms) * (1.0 / B)).reshape(1, 1)


def _tc_loss(eu, ei, label, W, b):
    return pl.pallas_call(
        _tc_loss_body,
        out_shape=jax.ShapeDtypeStruct((1, 1), jnp.float32),
    )(eu, ei, label, W, b)


def kernel(user, item, label, embed_user_w, embed_item_w, W, b):
    eu, ei = _sc_gather(user, item,
                        embed_user_w.astype(jnp.bfloat16),
                        embed_item_w.astype(jnp.bfloat16))
    loss = _tc_loss(eu, ei, label, W, b)
    return loss.reshape(())


# speedup vs baseline: 1.1421x; 1.1421x over previous
import functools

import jax
import jax.numpy as jnp
from jax import lax
from jax.experimental import pallas as pl
from jax.experimental.pallas import tpu as pltpu
from jax.experimental.pallas import tpu_sc as plsc

B = 16384
D = 32
PACK = 128 // D
NC = 2
NS = 16
NW = NC * NS
BPW = B // NW
CH = 128
NCHUNK = BPW // CH


def _sc_gather(uh, ih, uw4, iw4):
    mesh = plsc.VectorSubcoreMesh(core_axis_name="c", subcore_axis_name="s")

    @functools.partial(
        pl.kernel,
        mesh=mesh,
        compiler_params=pltpu.CompilerParams(use_tc_tiling_on_sc=True),
        out_type=(
            jax.ShapeDtypeStruct((B, 128), jnp.float32),
            jax.ShapeDtypeStruct((B, 128), jnp.float32),
        ),
        scratch_types=[
            pltpu.VMEM((CH,), jnp.int32),
            pltpu.VMEM((CH,), jnp.int32),
            pltpu.VMEM((CH,), jnp.int32),
            pltpu.VMEM((CH,), jnp.int32),
            pltpu.VMEM((BPW, 128), jnp.float32),
            pltpu.SemaphoreType.DMA,
        ],
    )
    def k(uw_hbm, iw_hbm, uidx_hbm, iidx_hbm, pu_hbm, pi_hbm,
          idx0, idx1, idx2, idx3, rows_v, sem):
        wid = lax.axis_index("s") * NC + lax.axis_index("c")
        base = wid * BPW
        idxbufs = [idx0, idx1, idx2, idx3]
        for tab_hbm, src_idx, out_hbm in ((uw_hbm, uidx_hbm, pu_hbm),
                                          (iw_hbm, iidx_hbm, pi_hbm)):
            copies = []
            for j in range(NCHUNK):
                pltpu.sync_copy(src_idx.at[wid, j], idxbufs[j])
                copies.append(pltpu.async_copy(
                    tab_hbm.at[idxbufs[j]], rows_v.at[pl.ds(j * CH, CH)], sem))
            for c in copies:
                c.wait()
            pltpu.sync_copy(rows_v, out_hbm.at[pl.ds(base, BPW)])

    return k(uw4, iw4, uh.reshape(NW, NCHUNK, CH), ih.reshape(NW, NCHUNK, CH))


def _tc_loss_body(pu_ref, pi_ref, su_ref, si_ref, lab_ref, w_ref, b_ref,
                  out_ref):
    pu = pu_ref[...]
    pi = pi_ref[...]
    su = su_ref[...].reshape(B, 1)
    si = si_ref[...].reshape(B, 1)
    eu = jnp.zeros((B, D), jnp.float32)
    ei = jnp.zeros((B, D), jnp.float32)
    for s in range(PACK):
        eu = jnp.where(su == s, pu[:, s * D:(s + 1) * D], eu)
        ei = jnp.where(si == s, pi[:, s * D:(s + 1) * D], ei)
    t = eu * ei * w_ref[...]
    x = jnp.sum(t, axis=1) + b_ref[0]
    y = lab_ref[...]
    terms = jnp.maximum(x, 0.0) - x * y + jnp.log1p(jnp.exp(-jnp.abs(x)))
    out_ref[...] = (jnp.sum(terms) * (1.0 / B)).reshape(1, 1)


def _tc_loss(pu, pi, su, si, label, W, b):
    return pl.pallas_call(
        _tc_loss_body,
        out_shape=jax.ShapeDtypeStruct((1, 1), jnp.float32),
    )(pu, pi, su, si, label, W, b)


def kernel(user, item, label, embed_user_w, embed_item_w, W, b):
    uw4 = embed_user_w.reshape(-1, 128)
    iw4 = embed_item_w.reshape(-1, 128)
    uh = user // PACK
    ih = item // PACK
    su = user % PACK
    si = item % PACK
    pu, pi = _sc_gather(uh, ih, uw4, iw4)
    loss = _tc_loss(pu, pi, su, si, label, W, b)
    return loss.reshape(())


# speedup vs baseline: 1.1660x; 1.0210x over previous
import functools

import jax
import jax.numpy as jnp
from jax import lax
from jax.experimental import pallas as pl
from jax.experimental.pallas import tpu as pltpu
from jax.experimental.pallas import tpu_sc as plsc

B = 16384
D = 32
NC = 2
NS = 16
NW = NC * NS
BPW = B // NW
CH = 128
NCHUNK = BPW // CH


def _sc_gather(user_idx, item_idx, embed_user_w, embed_item_w):
    mesh = plsc.VectorSubcoreMesh(core_axis_name="c", subcore_axis_name="s")

    @functools.partial(
        pl.kernel,
        mesh=mesh,
        compiler_params=pltpu.CompilerParams(use_tc_tiling_on_sc=False),
        out_type=(
            jax.ShapeDtypeStruct((B, D), jnp.float32),
            jax.ShapeDtypeStruct((B, D), jnp.float32),
        ),
        scratch_types=[
            pltpu.VMEM((NCHUNK, CH), jnp.int32),
            pltpu.VMEM((NCHUNK, CH), jnp.int32),
            pltpu.VMEM((BPW, D), jnp.float32),
            pltpu.VMEM((BPW, D), jnp.float32),
            pltpu.SemaphoreType.DMA,
            pltpu.SemaphoreType.DMA,
        ],
    )
    def k(uw_hbm, iw_hbm, uidx_hbm, iidx_hbm, eu_hbm, ei_hbm,
          uidx_v, iidx_v, urows_v, irows_v, sem_u, sem_i):
        wid = lax.axis_index("s") * NC + lax.axis_index("c")
        base = wid * BPW
        pltpu.sync_copy(uidx_hbm.at[wid], uidx_v)
        pltpu.sync_copy(iidx_hbm.at[wid], iidx_v)
        copies = []
        for j in range(NCHUNK):
            dst = pl.ds(j * CH, CH)
            copies.append(
                pltpu.async_copy(uw_hbm.at[uidx_v.at[j]], urows_v.at[dst], sem_u))
            copies.append(
                pltpu.async_copy(iw_hbm.at[iidx_v.at[j]], irows_v.at[dst], sem_i))
        for c in copies:
            c.wait()
        pltpu.sync_copy(urows_v, eu_hbm.at[pl.ds(base, BPW)])
        pltpu.sync_copy(irows_v, ei_hbm.at[pl.ds(base, BPW)])

    return k(embed_user_w, embed_item_w,
             user_idx.reshape(NW, NCHUNK, CH),
             item_idx.reshape(NW, NCHUNK, CH))


def _tc_loss_body(eu_ref, ei_ref, lab_ref, w_ref, b_ref, out_ref):
    t = eu_ref[...] * ei_ref[...] * w_ref[...]
    x = jnp.sum(t, axis=1) + b_ref[0]
    y = lab_ref[...]
    terms = jnp.maximum(x, 0.0) - x * y + jnp.log1p(jnp.exp(-jnp.abs(x)))
    out_ref[...] = (jnp.sum(terms) * (1.0 / B)).reshape(1, 1)


def _tc_loss(eu, ei, label, W, b):
    return pl.pallas_call(
        _tc_loss_body,
        out_shape=jax.ShapeDtypeStruct((1, 1), jnp.float32),
    )(eu, ei, label, W, b)


def kernel(user, item, label, embed_user_w, embed_item_w, W, b):
    eu, ei = _sc_gather(user, item, embed_user_w, embed_item_w)
    loss = _tc_loss(eu, ei, label, W, b)
    return loss.reshape(())
